# baseline (speedup 1.0000x reference)
import dataclasses
import functools

import jax
import jax.numpy as jnp
from jax import lax
from jax.experimental import pallas as pl
from jax.experimental.pallas import tpu as pltpu
from jax.experimental.pallas import tpu_sc as plsc

_MAX_REL = 50
_HIDDEN = 128
_VOCAB = 2 * _MAX_REL + 1
_NC, _NS = 2, 16
_NW = _NC * _NS
_CHUNK = 128
_LANES = 16


def _idx_body(s_ref, o_ref):
    s = s_ref[...]
    d = s[:, :, None] - s[:, None, :]
    o_ref[...] = jnp.clip(d, -_MAX_REL, _MAX_REL) + _MAX_REL


def _compute_indices(s):
    B, N = s.shape
    return pl.pallas_call(
        _idx_body,
        out_shape=jax.ShapeDtypeStruct((B, N, N), jnp.int32),
    )(s)


def _rep_body(t_ref, o_ref):
    o_ref[0] = t_ref[...]


def _replicate_table(table):
    return pl.pallas_call(
        _rep_body,
        grid=(_NW,),
        in_specs=[pl.BlockSpec((_VOCAB, _HIDDEN), lambda i: (0, 0))],
        out_specs=pl.BlockSpec((1, _VOCAB, _HIDDEN), lambda i: (i, 0, 0)),
        out_shape=jax.ShapeDtypeStruct((_NW, _VOCAB, _HIDDEN), jnp.float32),
    )(table)


def _sc_lookup(table_rep, idx_flat, num_idx):
    mesh = plsc.VectorSubcoreMesh(core_axis_name="c", subcore_axis_name="s")
    rows_per_w = num_idx // _NW
    n_chunks = rows_per_w // _CHUNK

    cp = pltpu.CompilerParams()
    if "needs_layout_passes" in pltpu.CompilerParams.__dataclass_fields__:
        cp = dataclasses.replace(cp, needs_layout_passes=False)

    @functools.partial(
        pl.kernel,
        out_type=jax.ShapeDtypeStruct((num_idx, _HIDDEN), jnp.float32),
        mesh=mesh,
        compiler_params=cp,
        scratch_types=[
            pltpu.VMEM((_CHUNK,), jnp.int32),
            pltpu.VMEM((_CHUNK,), jnp.int32),
            pltpu.VMEM((_CHUNK, _HIDDEN), jnp.float32),
            pltpu.VMEM((_CHUNK, _HIDDEN), jnp.float32),
            pltpu.SemaphoreType.DMA,
            pltpu.SemaphoreType.DMA,
            pltpu.SemaphoreType.DMA,
            pltpu.SemaphoreType.DMA,
        ],
    )
    def lookup_kernel(table_hbm, idx_hbm, out_hbm, idx_v0, idx_v1,
                      rows_v0, rows_v1, isem0, isem1, osem0, osem1):
        wid = lax.axis_index("s") * _NC + lax.axis_index("c")
        w_base = wid * rows_per_w
        woff = lax.broadcast(wid * _VOCAB, (_LANES,))

        def run(d):
            d.start()
            d.wait()

        def fetch_idx(c, idx_ref, sem):
            d = pltpu.make_async_copy(
                idx_hbm.at[pl.ds(w_base + c * _CHUNK, _CHUNK)], idx_ref, sem)
            run(d)
            for k in range(_CHUNK // _LANES):
                sl = pl.ds(k * _LANES, _LANES)
                idx_ref[sl] = idx_ref[sl] + woff

        def gather(idx_ref, rows_ref, sem):
            return pltpu.make_async_copy(table_hbm.at[idx_ref], rows_ref, sem)

        def drain(c, rows_ref, sem):
            return pltpu.make_async_copy(
                rows_ref, out_hbm.at[pl.ds(w_base + c * _CHUNK, _CHUNK)], sem)

        fetch_idx(0, idx_v0, isem0)
        run(gather(idx_v0, rows_v0, isem0))
        drain(0, rows_v0, osem0).start()
        fetch_idx(1, idx_v1, isem1)
        run(gather(idx_v1, rows_v1, isem1))
        drain(1, rows_v1, osem1).start()

        @pl.loop(1, n_chunks // 2)
        def _(p):
            c = 2 * p
            fetch_idx(c, idx_v0, isem0)
            drain(c - 2, rows_v0, osem0).wait()
            run(gather(idx_v0, rows_v0, isem0))
            drain(c, rows_v0, osem0).start()
            fetch_idx(c + 1, idx_v1, isem1)
            drain(c - 1, rows_v1, osem1).wait()
            run(gather(idx_v1, rows_v1, isem1))
            drain(c + 1, rows_v1, osem1).start()

        drain(n_chunks - 2, rows_v0, osem0).wait()
        drain(n_chunks - 1, rows_v1, osem1).wait()

    return lookup_kernel(table_rep, idx_flat)


def kernel(step_numbers, relative_embeddings):
    B, N = step_numbers.shape
    num_idx = B * N * N
    s = step_numbers.astype(jnp.int32)
    idx = _compute_indices(s)
    rep = _replicate_table(relative_embeddings).reshape(_NW * _VOCAB, _HIDDEN)
    out = _sc_lookup(rep, idx.reshape(num_idx), num_idx)
    return out.reshape(B, N, N, _HIDDEN)


# speedup vs baseline: 1.1950x; 1.1950x over previous
import dataclasses
import functools

import jax
import jax.numpy as jnp
from jax import lax
from jax.experimental import pallas as pl
from jax.experimental.pallas import tpu as pltpu
from jax.experimental.pallas import tpu_sc as plsc

_MAX_REL = 50
_HIDDEN = 128
_VOCAB = 2 * _MAX_REL + 1
_NC, _NS = 2, 16
_NW = _NC * _NS
_CHUNK = 128
_LANES = 16


def _idx_body(s_ref, o_ref):
    s = s_ref[...]
    d = s[:, :, None] - s[:, None, :]
    o_ref[...] = jnp.clip(d, -_MAX_REL, _MAX_REL) + _MAX_REL


def _compute_indices(s):
    B, N = s.shape
    return pl.pallas_call(
        _idx_body,
        out_shape=jax.ShapeDtypeStruct((B, N, N), jnp.int32),
    )(s)


def _rep_body(t_ref, o_ref):
    o_ref[...] = jnp.broadcast_to(
        t_ref[...][:, None, :], (_VOCAB, _NW, _HIDDEN))


def _replicate_table(table):
    return pl.pallas_call(
        _rep_body,
        out_shape=jax.ShapeDtypeStruct((_VOCAB, _NW, _HIDDEN), jnp.float32),
    )(table)


def _sc_lookup(table_rep, idx_flat, num_idx):
    mesh = plsc.VectorSubcoreMesh(core_axis_name="c", subcore_axis_name="s")
    rows_per_w = num_idx // _NW
    n_chunks = rows_per_w // _CHUNK

    cp = pltpu.CompilerParams()
    if "needs_layout_passes" in pltpu.CompilerParams.__dataclass_fields__:
        cp = dataclasses.replace(cp, needs_layout_passes=False)

    @functools.partial(
        pl.kernel,
        out_type=jax.ShapeDtypeStruct((num_idx, _HIDDEN), jnp.float32),
        mesh=mesh,
        compiler_params=cp,
        scratch_types=[
            pltpu.VMEM((_CHUNK,), jnp.int32),
            pltpu.VMEM((_CHUNK,), jnp.int32),
            pltpu.VMEM((_CHUNK, _HIDDEN), jnp.float32),
            pltpu.VMEM((_CHUNK, _HIDDEN), jnp.float32),
            pltpu.SemaphoreType.DMA,
            pltpu.SemaphoreType.DMA,
            pltpu.SemaphoreType.DMA,
            pltpu.SemaphoreType.DMA,
        ],
    )
    def lookup_kernel(table_hbm, idx_hbm, out_hbm, idx_v0, idx_v1,
                      rows_v0, rows_v1, isem0, isem1, osem0, osem1):
        wid = lax.axis_index("s") * _NC + lax.axis_index("c")
        w_base = wid * rows_per_w
        woff = lax.broadcast(wid, (_LANES,))

        def run(d):
            d.start()
            d.wait()

        def fetch_idx(c, idx_ref, sem):
            d = pltpu.make_async_copy(
                idx_hbm.at[pl.ds(w_base + c * _CHUNK, _CHUNK)], idx_ref, sem)
            run(d)
            for k in range(_CHUNK // _LANES):
                sl = pl.ds(k * _LANES, _LANES)
                idx_ref[sl] = idx_ref[sl] * _NW + woff

        def gather(idx_ref, rows_ref, sem):
            return pltpu.make_async_copy(table_hbm.at[idx_ref], rows_ref, sem)

        def drain(c, rows_ref, sem):
            return pltpu.make_async_copy(
                rows_ref, out_hbm.at[pl.ds(w_base + c * _CHUNK, _CHUNK)], sem)

        fetch_idx(0, idx_v0, isem0)
        run(gather(idx_v0, rows_v0, isem0))
        drain(0, rows_v0, osem0).start()
        fetch_idx(1, idx_v1, isem1)
        run(gather(idx_v1, rows_v1, isem1))
        drain(1, rows_v1, osem1).start()

        @pl.loop(1, n_chunks // 2)
        def _(p):
            c = 2 * p
            fetch_idx(c, idx_v0, isem0)
            drain(c - 2, rows_v0, osem0).wait()
            run(gather(idx_v0, rows_v0, isem0))
            drain(c, rows_v0, osem0).start()
            fetch_idx(c + 1, idx_v1, isem1)
            drain(c - 1, rows_v1, osem1).wait()
            run(gather(idx_v1, rows_v1, isem1))
            drain(c + 1, rows_v1, osem1).start()

        drain(n_chunks - 2, rows_v0, osem0).wait()
        drain(n_chunks - 1, rows_v1, osem1).wait()

    return lookup_kernel(table_rep, idx_flat)


def kernel(step_numbers, relative_embeddings):
    B, N = step_numbers.shape
    num_idx = B * N * N
    s = step_numbers.astype(jnp.int32)
    idx = _compute_indices(s)
    rep = _replicate_table(relative_embeddings).reshape(_NW * _VOCAB, _HIDDEN)
    out = _sc_lookup(rep, idx.reshape(num_idx), num_idx)
    return out.reshape(B, N, N, _HIDDEN)


# speedup vs baseline: 2.1340x; 1.7857x over previous
import dataclasses
import functools

import jax
import jax.numpy as jnp
from jax import lax
from jax.experimental import pallas as pl
from jax.experimental.pallas import tpu as pltpu
from jax.experimental.pallas import tpu_sc as plsc

_MAX_REL = 50
_HIDDEN = 128
_VOCAB = 2 * _MAX_REL + 1
_TBL = _VOCAB * _HIDDEN
_NC, _NS = 2, 16
_NW = _NC * _NS
_CHUNK = 128
_LANES = 16
_NE, _NSTR, _ITERS = 6, 2, 16
_GVIEW = _TBL - _HIDDEN + _LANES


def _idx_body(s_ref, o_ref):
    s = s_ref[...]
    d = s[:, :, None] - s[:, None, :]
    o_ref[...] = jnp.clip(d, -_MAX_REL, _MAX_REL) + _MAX_REL


def _compute_indices(s):
    B, N = s.shape
    return pl.pallas_call(
        _idx_body,
        out_shape=jax.ShapeDtypeStruct((B, N, N), jnp.int32),
    )(s)


def _rep_body(t_ref, o_ref):
    o_ref[...] = jnp.broadcast_to(
        t_ref[...][:, None, :], (_VOCAB, _NW, _HIDDEN))


def _replicate_table(table):
    return pl.pallas_call(
        _rep_body,
        out_shape=jax.ShapeDtypeStruct((_VOCAB, _NW, _HIDDEN), jnp.float32),
    )(table)


def _sc_lookup(table, table_rep, idx_flat, num_idx):
    mesh = plsc.VectorSubcoreMesh(core_axis_name="c", subcore_axis_name="s")
    rows_per_w = num_idx // _NW
    n_chunks = rows_per_w // _CHUNK
    assert n_chunks == (_NE + _NSTR) * _ITERS
    e_rows = _NE * _ITERS * _CHUNK
    out_elems = num_idx * _HIDDEN

    cp = pltpu.CompilerParams()
    if "needs_layout_passes" in pltpu.CompilerParams.__dataclass_fields__:
        cp = dataclasses.replace(cp, needs_layout_passes=False)

    @functools.partial(
        pl.kernel,
        out_type=jax.ShapeDtypeStruct((num_idx, _HIDDEN), jnp.float32),
        mesh=mesh,
        compiler_params=cp,
        scratch_types=[
            pltpu.VMEM((_TBL,), jnp.float32),
            pltpu.VMEM((_CHUNK,), jnp.int32),
            pltpu.VMEM((_CHUNK,), jnp.int32),
            pltpu.VMEM((_CHUNK,), jnp.int32),
            pltpu.VMEM((_CHUNK, _HIDDEN), jnp.float32),
            pltpu.VMEM((_CHUNK, _HIDDEN), jnp.float32),
            pltpu.VMEM((_CHUNK, _HIDDEN), jnp.float32),
            pltpu.VMEM((_CHUNK, _HIDDEN), jnp.float32),
            pltpu.VMEM((_CHUNK, _HIDDEN), jnp.float32),
            pltpu.SemaphoreType.DMA,
            pltpu.SemaphoreType.DMA,
            pltpu.SemaphoreType.DMA,
            pltpu.SemaphoreType.DMA,
            pltpu.SemaphoreType.DMA,
            pltpu.SemaphoreType.DMA,
            pltpu.SemaphoreType.DMA,
            pltpu.SemaphoreType.DMA,
        ],
    )
    def lookup_kernel(table_hbm, rep_hbm, idx_hbm, out_hbm,
                      table_v, idx_e, idx_sa, idx_sb,
                      eb0, eb1, eb2, sba, sbb,
                      esem0, esem1, esem2, gsema, gsemb,
                      ssema, ssemb, tsem):
        wid = lax.axis_index("s") * _NC + lax.axis_index("c")
        w_base = wid * rows_per_w
        e_base = w_base
        s_base = w_base + e_rows
        woff = lax.broadcast(wid, (_LANES,))

        pltpu.async_copy(table_hbm, table_v, tsem).wait()

        col = lax.iota(jnp.int32, _LANES)
        views = [table_v.at[pl.ds(g * _LANES, _GVIEW)]
                 for g in range(_HIDDEN // _LANES)]

        def expand_fill(c, buf_ref):
            pltpu.async_copy(
                idx_hbm.at[pl.ds(e_base + c * _CHUNK, _CHUNK)], idx_e,
                tsem).wait()
            for k in range(_CHUNK // _LANES):
                sl = pl.ds(k * _LANES, _LANES)
                idx_e[sl] = idx_e[sl] * _HIDDEN

            @pl.loop(0, _CHUNK, step=16)
            def _(r0):
                for rr in range(16):
                    base = plsc.load_gather(
                        idx_e, [lax.broadcast(r0 + rr, (_LANES,))])
                    addr0 = base + col
                    for g in range(_HIDDEN // _LANES):
                        v = plsc.load_gather(views[g], [addr0])
                        buf_ref[r0 + rr, pl.ds(g * _LANES, _LANES)] = v

        def expand_drain(c, buf_ref, sem):
            return pltpu.make_async_copy(
                buf_ref,
                out_hbm.at[pl.ds(e_base + c * _CHUNK, _CHUNK)], sem)

        def stream_prep(c, idx_ref):
            pltpu.async_copy(
                idx_hbm.at[pl.ds(s_base + c * _CHUNK, _CHUNK)], idx_ref,
                tsem).wait()
            for k in range(_CHUNK // _LANES):
                sl = pl.ds(k * _LANES, _LANES)
                idx_ref[sl] = idx_ref[sl] * _NW + woff

        def stream_gather(idx_ref, buf_ref, sem):
            return pltpu.make_async_copy(rep_hbm.at[idx_ref], buf_ref, sem)

        def stream_drain(c, buf_ref, sem):
            return pltpu.make_async_copy(
                buf_ref,
                out_hbm.at[pl.ds(s_base + c * _CHUNK, _CHUNK)], sem)

        ebufs = (eb0, eb1, eb2)
        esems = (esem0, esem1, esem2)

        @pl.loop(0, _ITERS)
        def _(t):
            sc_a = 2 * t
            stream_prep(sc_a, idx_sa)

            @pl.when(t > 0)
            def _():
                stream_drain(sc_a - 2, sba, ssema).wait()
            g_a = stream_gather(idx_sa, sba, gsema)
            g_a.start()

            for k in range(3):
                c = _NE * t + k

                @pl.when(t > 0)
                def _():
                    expand_drain(c - _NE, ebufs[k], esems[k]).wait()
                expand_fill(c, ebufs[k])
                expand_drain(c, ebufs[k], esems[k]).start()

            g_a.wait()
            stream_drain(sc_a, sba, ssema).start()

            sc_b = 2 * t + 1
            stream_prep(sc_b, idx_sb)

            @pl.when(t > 0)
            def _():
                stream_drain(sc_b - 2, sbb, ssemb).wait()
            g_b = stream_gather(idx_sb, sbb, gsemb)
            g_b.start()

            for k in range(3):
                c = _NE * t + 3 + k
                expand_drain(c - 3, ebufs[k], esems[k]).wait()
                expand_fill(c, ebufs[k])
                expand_drain(c, ebufs[k], esems[k]).start()

            g_b.wait()
            stream_drain(sc_b, sbb, ssemb).start()

        for k in range(3):
            expand_drain(_NE * (_ITERS - 1) + 3 + k, ebufs[k],
                         esems[k]).wait()
        stream_drain(2 * _ITERS - 2, sba, ssema).wait()
        stream_drain(2 * _ITERS - 1, sbb, ssemb).wait()

    return lookup_kernel(table.reshape(_TBL), table_rep, idx_flat)


def kernel(step_numbers, relative_embeddings):
    B, N = step_numbers.shape
    num_idx = B * N * N
    s = step_numbers.astype(jnp.int32)
    idx = _compute_indices(s)
    rep = _replicate_table(relative_embeddings).reshape(_NW * _VOCAB, _HIDDEN)
    out = _sc_lookup(relative_embeddings, rep, idx.reshape(num_idx), num_idx)
    return out.reshape(B, N, N, _HIDDEN)


# speedup vs baseline: 2.1384x; 1.0021x over previous
import dataclasses
import functools

import jax
import jax.numpy as jnp
from jax import lax
from jax.experimental import pallas as pl
from jax.experimental.pallas import tpu as pltpu
from jax.experimental.pallas import tpu_sc as plsc

_MAX_REL = 50
_HIDDEN = 128
_VOCAB = 2 * _MAX_REL + 1
_TBL = _VOCAB * _HIDDEN
_NC, _NS = 2, 16
_NW = _NC * _NS
_CHUNK = 128
_LANES = 16
_NE, _NSTR, _ITERS = 18, 14, 4
_EPAT = (2, 1, 1, 2, 1, 1, 1) * 2
_GVIEW = _TBL - _HIDDEN + _LANES


def _idx_body(s_ref, o_ref):
    s = s_ref[...]
    d = s[:, :, None] - s[:, None, :]
    o_ref[...] = jnp.clip(d, -_MAX_REL, _MAX_REL) + _MAX_REL


def _compute_indices(s):
    B, N = s.shape
    return pl.pallas_call(
        _idx_body,
        out_shape=jax.ShapeDtypeStruct((B, N, N), jnp.int32),
    )(s)


def _rep_body(t_ref, o_ref):
    o_ref[...] = jnp.broadcast_to(
        t_ref[...][:, None, :], (_VOCAB, _NW, _HIDDEN))


def _replicate_table(table):
    return pl.pallas_call(
        _rep_body,
        out_shape=jax.ShapeDtypeStruct((_VOCAB, _NW, _HIDDEN), jnp.float32),
    )(table)


def _sc_lookup(table, table_rep, idx_flat, num_idx):
    mesh = plsc.VectorSubcoreMesh(core_axis_name="c", subcore_axis_name="s")
    rows_per_w = num_idx // _NW
    n_chunks = rows_per_w // _CHUNK
    assert n_chunks == (_NE + _NSTR) * _ITERS
    e_rows = _NE * _ITERS * _CHUNK
    out_elems = num_idx * _HIDDEN

    cp = pltpu.CompilerParams()
    if "needs_layout_passes" in pltpu.CompilerParams.__dataclass_fields__:
        cp = dataclasses.replace(cp, needs_layout_passes=False)

    @functools.partial(
        pl.kernel,
        out_type=jax.ShapeDtypeStruct((num_idx, _HIDDEN), jnp.float32),
        mesh=mesh,
        compiler_params=cp,
        scratch_types=[
            pltpu.VMEM((_TBL,), jnp.float32),
            pltpu.VMEM((_CHUNK,), jnp.int32),
            pltpu.VMEM((_CHUNK,), jnp.int32),
            pltpu.VMEM((_CHUNK,), jnp.int32),
            pltpu.VMEM((_CHUNK, _HIDDEN), jnp.float32),
            pltpu.VMEM((_CHUNK, _HIDDEN), jnp.float32),
            pltpu.VMEM((_CHUNK, _HIDDEN), jnp.float32),
            pltpu.VMEM((_CHUNK, _HIDDEN), jnp.float32),
            pltpu.VMEM((_CHUNK, _HIDDEN), jnp.float32),
            pltpu.SemaphoreType.DMA,
            pltpu.SemaphoreType.DMA,
            pltpu.SemaphoreType.DMA,
            pltpu.SemaphoreType.DMA,
            pltpu.SemaphoreType.DMA,
            pltpu.SemaphoreType.DMA,
            pltpu.SemaphoreType.DMA,
            pltpu.SemaphoreType.DMA,
        ],
    )
    def lookup_kernel(table_hbm, rep_hbm, idx_hbm, out_hbm,
                      table_v, idx_e, idx_sa, idx_sb,
                      eb0, eb1, eb2, sba, sbb,
                      esem0, esem1, esem2, gsema, gsemb,
                      ssema, ssemb, tsem):
        wid = lax.axis_index("s") * _NC + lax.axis_index("c")
        w_base = wid * rows_per_w
        e_base = w_base
        s_base = w_base + e_rows
        woff = lax.broadcast(wid, (_LANES,))

        pltpu.async_copy(table_hbm, table_v, tsem).wait()

        col = lax.iota(jnp.int32, _LANES)
        views = [table_v.at[pl.ds(g * _LANES, _GVIEW)]
                 for g in range(_HIDDEN // _LANES)]

        def expand_fill(c, buf_ref):
            pltpu.async_copy(
                idx_hbm.at[pl.ds(e_base + c * _CHUNK, _CHUNK)], idx_e,
                tsem).wait()
            for k in range(_CHUNK // _LANES):
                sl = pl.ds(k * _LANES, _LANES)
                idx_e[sl] = idx_e[sl] * _HIDDEN

            @pl.loop(0, _CHUNK, step=16)
            def _(r0):
                for rr in range(16):
                    base = plsc.load_gather(
                        idx_e, [lax.broadcast(r0 + rr, (_LANES,))])
                    addr0 = base + col
                    for g in range(_HIDDEN // _LANES):
                        v = plsc.load_gather(views[g], [addr0])
                        buf_ref[r0 + rr, pl.ds(g * _LANES, _LANES)] = v

        def expand_drain(c, buf_ref, sem):
            return pltpu.make_async_copy(
                buf_ref,
                out_hbm.at[pl.ds(e_base + c * _CHUNK, _CHUNK)], sem)

        def stream_prep(c, idx_ref):
            pltpu.async_copy(
                idx_hbm.at[pl.ds(s_base + c * _CHUNK, _CHUNK)], idx_ref,
                tsem).wait()
            for k in range(_CHUNK // _LANES):
                sl = pl.ds(k * _LANES, _LANES)
                idx_ref[sl] = idx_ref[sl] * _NW + woff

        def stream_gather(idx_ref, buf_ref, sem):
            return pltpu.make_async_copy(rep_hbm.at[idx_ref], buf_ref, sem)

        def stream_drain(c, buf_ref, sem):
            return pltpu.make_async_copy(
                buf_ref,
                out_hbm.at[pl.ds(s_base + c * _CHUNK, _CHUNK)], sem)

        ebufs = (eb0, eb1, eb2)
        esems = (esem0, esem1, esem2)
        sbufs = (sba, sbb)
        gsems = (gsema, gsemb)
        ssems = (ssema, ssemb)

        @pl.loop(0, _ITERS)
        def _(t):
            e_done = 0
            for q in range(_NSTR):
                p = q % 2
                s_c = _NSTR * t + q

                def bwait(s_c=s_c, p=p):
                    stream_drain(s_c - 2, sbufs[p], ssems[p]).wait()
                if q >= 2:
                    bwait()
                else:
                    pl.when(t > 0)(bwait)

                stream_prep(s_c, (idx_sa, idx_sb)[p])
                stream_gather((idx_sa, idx_sb)[p], sbufs[p], gsems[p]).start()

                for _k in range(_EPAT[q]):
                    c = _NE * t + e_done
                    m = e_done % 3

                    def ewait(c=c, m=m):
                        expand_drain(c - 3, ebufs[m], esems[m]).wait()
                    if e_done >= 3:
                        ewait()
                    else:
                        pl.when(t > 0)(ewait)
                    expand_fill(c, ebufs[m])
                    expand_drain(c, ebufs[m], esems[m]).start()
                    e_done += 1

                def gwait(s_c=s_c, p=p):
                    op = 1 - p
                    stream_gather((idx_sa, idx_sb)[op], sbufs[op],
                                  gsems[op]).wait()
                    stream_drain(s_c - 1, sbufs[op], ssems[op]).start()
                if q >= 1:
                    gwait()
                else:
                    pl.when(t > 0)(gwait)

        last = _NSTR * _ITERS - 1
        stream_gather(idx_sb, sbb, gsemb).wait()
        stream_drain(last, sbb, ssemb).start()
        stream_drain(last - 1, sba, ssema).wait()
        stream_drain(last, sbb, ssemb).wait()
        for k in range(3):
            c = _NE * _ITERS - 3 + k
            expand_drain(c, ebufs[c % 3], esems[c % 3]).wait()

    return lookup_kernel(table.reshape(_TBL), table_rep, idx_flat)


def kernel(step_numbers, relative_embeddings):
    B, N = step_numbers.shape
    num_idx = B * N * N
    s = step_numbers.astype(jnp.int32)
    idx = _compute_indices(s)
    rep = _replicate_table(relative_embeddings).reshape(_NW * _VOCAB, _HIDDEN)
    out = _sc_lookup(relative_embeddings, rep, idx.reshape(num_idx), num_idx)
    return out.reshape(B, N, N, _HIDDEN)
